# initial kernel scaffold (unmeasured)
import jax
import jax.numpy as jnp
from jax import lax
from jax.experimental import pallas as pl
from jax.experimental.pallas import tpu as pltpu

N_DEV = 4
SQ = 1024
D = 1024
HG = 8
DH = 128
SCALE = 0.08838834764831843


def _body(x_ref, wq_ref, wo_ref, k_ref, v_ref, out_ref, comm, ctx, ssem, rsem):
    my = lax.axis_index("i")
    left = lax.rem(my + N_DEV - 1, N_DEV)
    right = lax.rem(my + 1, N_DEV)

    comm[0, :D, :] = wq_ref[...]
    comm[0, D:, :] = wo_ref[...]

    qb = lax.broadcasted_iota(jnp.int32, (SQ, SQ), 0) // 64
    kb = lax.broadcasted_iota(jnp.int32, (SQ, SQ), 1) // 64
    maskadd = jnp.where((qb % 4) == (kb % 4), 0.0, -1e9).astype(jnp.float32)

    bsem = pltpu.get_barrier_semaphore()
    for nbr in (left, right):
        pl.semaphore_signal(
            bsem, inc=1, device_id=(nbr,), device_id_type=pl.DeviceIdType.MESH
        )
    pl.semaphore_wait(bsem, 2)

    for h in range(N_DEV):
        if h < N_DEV - 1:
            rdma = pltpu.make_async_remote_copy(
                src_ref=comm.at[h],
                dst_ref=comm.at[h + 1],
                send_sem=ssem.at[h],
                recv_sem=rsem.at[h],
                device_id=(right,),
                device_id_type=pl.DeviceIdType.MESH,
            )
            rdma.start()

        g = lax.rem(my + N_DEV - h, N_DEV)
        wq_h = comm[h, :D, :]
        wo_h = comm[h, D:, :]
        q_g = (
            jnp.dot(x_ref[...], wq_h, preferred_element_type=jnp.float32) * SCALE
        ).astype(jnp.bfloat16)

        def head(hh, _, g=g, q_g=q_g, maskadd=maskadd):
            q_h = lax.dynamic_slice(q_g, (0, hh * DH), (SQ, DH))
            k_h = pl.load(
                k_ref, (pl.ds(g * HG + hh, 1), slice(None), slice(None))
            ).reshape(SQ, DH)
            v_h = pl.load(
                v_ref, (pl.ds(g * HG + hh, 1), slice(None), slice(None))
            ).reshape(SQ, DH)
            scores = lax.dot_general(
                q_h, k_h, (((1,), (1,)), ((), ())),
                preferred_element_type=jnp.float32,
            )
            w = jnp.exp(scores + maskadd)
            wsum = jnp.sum(w, axis=1, keepdims=True)
            ctx_h = jnp.dot(
                w.astype(jnp.bfloat16), v_h, preferred_element_type=jnp.float32
            )
            ctx_h = ctx_h / wsum
            pl.store(
                ctx, (slice(None), pl.ds(hh * DH, DH)), ctx_h.astype(jnp.bfloat16)
            )
            return 0

        lax.fori_loop(0, HG, head, 0)

        part = jnp.dot(ctx[...], wo_h, preferred_element_type=jnp.float32)
        if h == 0:
            out_ref[...] = part
        else:
            out_ref[...] = out_ref[...] + part

        if h < N_DEV - 1:
            rdma.wait()


def kernel(x, Wq, K_ext, V_ext, Wo):
    my = lax.axis_index("i")
    xb = x[0].astype(jnp.bfloat16)
    wq = Wq.astype(jnp.bfloat16)
    wo = Wo.astype(jnp.bfloat16)
    k = lax.dynamic_index_in_dim(K_ext, my, 0, keepdims=False)
    v = lax.dynamic_index_in_dim(V_ext, my, 0, keepdims=False)
    k = jnp.transpose(k, (1, 0, 2)).astype(jnp.bfloat16)
    v = jnp.transpose(v, (1, 0, 2)).astype(jnp.bfloat16)

    out = pl.pallas_call(
        _body,
        out_shape=jax.ShapeDtypeStruct((SQ, D), jnp.float32),
        in_specs=[pl.BlockSpec(memory_space=pltpu.VMEM)] * 5,
        out_specs=pl.BlockSpec(memory_space=pltpu.VMEM),
        scratch_shapes=[
            pltpu.VMEM((N_DEV, 2 * D, D), jnp.bfloat16),
            pltpu.VMEM((SQ, HG * DH), jnp.bfloat16),
            pltpu.SemaphoreType.DMA((N_DEV - 1,)),
            pltpu.SemaphoreType.DMA((N_DEV - 1,)),
        ],
        compiler_params=pltpu.CompilerParams(collective_id=0),
    )(xb, wq, wo, k, v)
    return out[None, :, :]


# baseline (device time: 206016 ns/iter reference)
import jax
import jax.numpy as jnp
from jax import lax
from jax.experimental import pallas as pl
from jax.experimental.pallas import tpu as pltpu

N_DEV = 4
SQ = 1024
D = 1024
HG = 8
DH = 128
SCALE = 0.08838834764831843
CHUNK = 256


def _body(
    x_ref, wq_ref, wo_ref, k_ref, v_ref, out_ref, comm, ctx, qbuf, ssem, rsem
):
    my = lax.axis_index("i")
    left = lax.rem(my + N_DEV - 1, N_DEV)
    right = lax.rem(my + 1, N_DEV)

    comm[0, :D, :] = wq_ref[...]
    comm[0, D:, :] = wo_ref[...]

    qb = lax.broadcasted_iota(jnp.int32, (CHUNK, SQ), 0) // 64
    kb = lax.broadcasted_iota(jnp.int32, (CHUNK, SQ), 1) // 64
    maskadd = jnp.where((qb % 4) == (kb % 4), 0.0, -1e9).astype(jnp.float32)

    bsem = pltpu.get_barrier_semaphore()
    for nbr in (left, right):
        pl.semaphore_signal(
            bsem, inc=1, device_id=(nbr,), device_id_type=pl.DeviceIdType.MESH
        )
    pl.semaphore_wait(bsem, 2)

    for h in range(N_DEV):
        if h < N_DEV - 1:
            rdma = pltpu.make_async_remote_copy(
                src_ref=comm.at[h],
                dst_ref=comm.at[h + 1],
                send_sem=ssem.at[h],
                recv_sem=rsem.at[h],
                device_id=(right,),
                device_id_type=pl.DeviceIdType.MESH,
            )
            rdma.start()

        g = lax.rem(my + N_DEV - h, N_DEV)
        wq_h = comm[h, :D, :]
        wo_h = comm[h, D:, :]
        def chunk(c, _, wq_h=wq_h, g=g, maskadd=maskadd):
            qbuf[...] = (
                jnp.dot(
                    x_ref[pl.ds(c * CHUNK, CHUNK), :],
                    wq_h,
                    preferred_element_type=jnp.float32,
                )
                * SCALE
            ).astype(jnp.bfloat16)

            def head(hh, _):
                q_h = qbuf[:, pl.ds(hh * DH, DH)]
                k_h = k_ref[pl.ds(g * HG + hh, 1), :, :].reshape(SQ, DH)
                v_h = v_ref[pl.ds(g * HG + hh, 1), :, :].reshape(SQ, DH)
                scores = lax.dot_general(
                    q_h, k_h, (((1,), (1,)), ((), ())),
                    preferred_element_type=jnp.float32,
                )
                w = jnp.exp(scores + maskadd)
                wsum = jnp.sum(w, axis=1, keepdims=True)
                ctx_h = jnp.dot(
                    w.astype(jnp.bfloat16), v_h,
                    preferred_element_type=jnp.float32,
                )
                ctx_h = ctx_h / wsum
                ctx[pl.ds(c * CHUNK, CHUNK), pl.ds(hh * DH, DH)] = ctx_h.astype(
                    jnp.bfloat16
                )
                return 0

            lax.fori_loop(0, HG, head, 0)
            return 0

        lax.fori_loop(0, SQ // CHUNK, chunk, 0)

        part = jnp.dot(ctx[...], wo_h, preferred_element_type=jnp.float32)
        if h == 0:
            out_ref[...] = part
        else:
            out_ref[...] = out_ref[...] + part

        if h < N_DEV - 1:
            rdma.wait()


def kernel(x, Wq, K_ext, V_ext, Wo):
    my = lax.axis_index("i")
    xb = x[0].astype(jnp.bfloat16)
    wq = Wq.astype(jnp.bfloat16)
    wo = Wo.astype(jnp.bfloat16)
    k = lax.dynamic_index_in_dim(K_ext, my, 0, keepdims=False)
    v = lax.dynamic_index_in_dim(V_ext, my, 0, keepdims=False)
    k = jnp.transpose(k, (1, 0, 2)).astype(jnp.bfloat16)
    v = jnp.transpose(v, (1, 0, 2)).astype(jnp.bfloat16)

    out = pl.pallas_call(
        _body,
        out_shape=jax.ShapeDtypeStruct((SQ, D), jnp.float32),
        in_specs=[pl.BlockSpec(memory_space=pltpu.VMEM)] * 5,
        out_specs=pl.BlockSpec(memory_space=pltpu.VMEM),
        scratch_shapes=[
            pltpu.VMEM((N_DEV, 2 * D, D), jnp.bfloat16),
            pltpu.VMEM((SQ, HG * DH), jnp.bfloat16),
            pltpu.VMEM((CHUNK, HG * DH), jnp.bfloat16),
            pltpu.SemaphoreType.DMA((N_DEV - 1,)),
            pltpu.SemaphoreType.DMA((N_DEV - 1,)),
        ],
        compiler_params=pltpu.CompilerParams(collective_id=0),
    )(xb, wq, wo, k, v)
    return out[None, :, :]


# device time: 86628 ns/iter; 2.3782x vs baseline; 2.3782x over previous
import os

import jax
import jax.numpy as jnp
from jax import lax
from jax.experimental import pallas as pl
from jax.experimental.pallas import tpu as pltpu

_ABLATE = os.environ.get("ABLATE", "")

N_DEV = 4
SQ = 1024
D = 1024
HG = 8
DH = 128
SCALE = 0.08838834764831843
NR = 4
RG = SQ // NR
NB = 4
BLK = 64


def _body(
    x_ref, wq_ref, wo_ref, k_hbm, v_hbm, out_ref,
    xbf, own_wq, own_wo, cwr, ccwr, kslot, vslot, ctxb, qbuf,
    ssem, rsem, ksem, vsem,
):
    my = lax.axis_index("i")
    left = lax.rem(my + N_DEV - 1, N_DEV)
    right = lax.rem(my + 1, N_DEV)

    def stage_kv(g):
        if _ABLATE in ("nocompute", "nokv"):
            return []
        copies = []
        for hh in range(HG):
            for hbm, slot, sem in ((k_hbm, kslot, ksem), (v_hbm, vslot, vsem)):
                c = pltpu.make_async_copy(
                    hbm.at[my, :, :, :, g * HG + hh, :], slot.at[hh], sem.at[hh]
                )
                c.start()
                copies.append(c)
        return copies

    def ctx_compute(wq_g, kv_copies, slot):
        for c in kv_copies:
            c.wait()

        def chunk(r, _):
            qbuf[...] = (
                jnp.dot(
                    xbf[pl.ds(r * RG, RG), :],
                    wq_g,
                    preferred_element_type=jnp.float32,
                )
                * SCALE
            ).astype(jnp.bfloat16)

            def one_head(hh):
                q_h = qbuf[:, pl.ds(hh * DH, DH)]
                if _ABLATE == "nokv":
                    k_h = v_h = q_h
                else:
                    k_h = kslot[pl.ds(hh, 1), :, r, :, :].reshape(RG, DH).astype(
                        jnp.bfloat16
                    )
                    v_h = vslot[pl.ds(hh, 1), :, r, :, :].reshape(RG, DH).astype(
                        jnp.bfloat16
                    )
                scores = lax.dot_general(
                    q_h, k_h, (((1,), (1,)), ((), ())),
                    preferred_element_type=jnp.float32,
                )
                w = jnp.exp(scores)
                wsum = jnp.sum(w, axis=1, keepdims=True)
                ctx_h = jnp.dot(
                    w.astype(jnp.bfloat16), v_h,
                    preferred_element_type=jnp.float32,
                )
                ctx_h = ctx_h / wsum
                ctxb[slot, pl.ds(r * RG, RG), pl.ds(hh * DH, DH)] = (
                    ctx_h.astype(jnp.bfloat16)
                )

            for hh in range(HG):
                one_head(hh)
            return 0

        lax.fori_loop(0, NR, chunk, 0)

    def proj(wo_g, slot, first):
        for r in range(NR):
            part = jnp.dot(
                ctxb[slot, pl.ds(r * RG, RG), :],
                wo_g,
                preferred_element_type=jnp.float32,
            )
            for m in range(NB):
                rows = pl.ds((NR * m + r) * BLK, BLK)
                blk = part[m * BLK:(m + 1) * BLK, :]
                out_ref[rows, :] = blk if first else out_ref[rows, :] + blk

    for c in range(NR):
        own_wq[pl.ds(c * RG, RG), :] = wq_ref[pl.ds(c * RG, RG), :].astype(
            jnp.bfloat16
        )

    kv0 = stage_kv(my)

    comm = _ABLATE not in ("nocomm", "nokv")
    docompute = _ABLATE != "nocompute"

    if comm:
        bsem = pltpu.get_barrier_semaphore()
        for nbr in (left, right):
            pl.semaphore_signal(
                bsem, inc=1, device_id=(nbr,), device_id_type=pl.DeviceIdType.MESH
            )
        pl.semaphore_wait(bsem, 2)

    def rdma(i, src, dst, dev):
        return pltpu.make_async_remote_copy(
            src_ref=src, dst_ref=dst, send_sem=ssem.at[i], recv_sem=rsem.at[i],
            device_id=(dev,), device_id_type=pl.DeviceIdType.MESH,
        )

    flows = []
    if comm:
        r_cw_wq = rdma(0, own_wq, cwr.at[0], right)
        r_ccw_wq = rdma(1, own_wq, ccwr.at[0], left)
        r_cw_wq.start()
        r_ccw_wq.start()
        flows += [r_cw_wq, r_ccw_wq]

    for c in range(NR):
        own_wo[pl.ds(c * RG, RG), :] = wo_ref[pl.ds(c * RG, RG), :].astype(
            jnp.bfloat16
        )
    if comm:
        r_cw_wo = rdma(2, own_wo, cwr.at[1], right)
        r_cw_wo.start()
        flows.append(r_cw_wo)
    for r in range(NR):
        for m in range(NB):
            xbf[pl.ds((r * NB + m) * BLK, BLK), :] = x_ref[
                0, pl.ds((NR * m + r) * BLK, BLK), :
            ].astype(jnp.bfloat16)

    if docompute:
        ctx_compute(own_wq[...], kv0, 0)
        proj(own_wo[...], 0, True)
    else:
        out_ref[...] = jnp.zeros((SQ, D), jnp.float32)

    if comm:
        r_ccw_wq.wait_recv()
        r_ccw_fwd = rdma(3, ccwr.at[0], ccwr.at[1], left)
        r_ccw_wo = rdma(4, own_wo, ccwr.at[2], left)
        r_ccw_fwd.start()
        r_ccw_wo.start()
        flows += [r_ccw_fwd, r_ccw_wo]

        if docompute:
            kv1 = stage_kv(right)
            ctx_compute(ccwr[0], kv1, 2)
        r_cw_wq.wait_recv()
        if docompute:
            kv2 = stage_kv(left)
            ctx_compute(cwr[0], kv2, 1)

        r_cw_wo.wait_recv()
        r_cw_fwd = rdma(5, cwr.at[1], cwr.at[2], right)
        r_cw_fwd.start()
        flows.append(r_cw_fwd)
        if docompute:
            proj(cwr[1], 1, False)

        r_ccw_fwd.wait_recv()
        if docompute:
            kv3 = stage_kv(lax.rem(my + 2, N_DEV))
            ctx_compute(ccwr[1], kv3, 0)
        r_cw_fwd.wait_recv()
        if docompute:
            proj(cwr[2], 0, False)
        r_ccw_wo.wait_recv()
        if docompute:
            proj(ccwr[2], 2, False)

        for f in flows:
            f.wait_send()


def kernel(x, Wq, K_ext, V_ext, Wo):
    kv_shape = (N_DEV, NB, NR, BLK, 4 * HG, DH)
    out = pl.pallas_call(
        _body,
        out_shape=jax.ShapeDtypeStruct((SQ, D), jnp.float32),
        in_specs=[
            pl.BlockSpec(memory_space=pltpu.VMEM),
            pl.BlockSpec(memory_space=pltpu.VMEM),
            pl.BlockSpec(memory_space=pltpu.VMEM),
            pl.BlockSpec(memory_space=pl.ANY),
            pl.BlockSpec(memory_space=pl.ANY),
        ],
        out_specs=pl.BlockSpec(memory_space=pltpu.VMEM),
        scratch_shapes=[
            pltpu.VMEM((SQ, D), jnp.bfloat16),
            pltpu.VMEM((D, D), jnp.bfloat16),
            pltpu.VMEM((D, D), jnp.bfloat16),
            pltpu.VMEM((3, D, D), jnp.bfloat16),
            pltpu.VMEM((3, D, D), jnp.bfloat16),
            pltpu.VMEM((HG, NB, NR, BLK, DH), jnp.float32),
            pltpu.VMEM((HG, NB, NR, BLK, DH), jnp.float32),
            pltpu.VMEM((3, SQ, HG * DH), jnp.bfloat16),
            pltpu.VMEM((RG, HG * DH), jnp.bfloat16),
            pltpu.SemaphoreType.DMA((6,)),
            pltpu.SemaphoreType.DMA((6,)),
            pltpu.SemaphoreType.DMA((HG,)),
            pltpu.SemaphoreType.DMA((HG,)),
        ],
        compiler_params=pltpu.CompilerParams(
            vmem_limit_bytes=44 * 1024 * 1024,
            **(
                {}
                if _ABLATE in ("nocomm", "nokv")
                else {"collective_id": 0}
            ),
        ),
    )(x, Wq, Wo, K_ext.reshape(kv_shape), V_ext.reshape(kv_shape))
    return out[None, :, :]
